# baseline (speedup 1.0000x reference)
import functools

import jax
import jax.numpy as jnp
from jax import lax
from jax.experimental import pallas as pl
from jax.experimental.pallas import tpu as pltpu
from jax.experimental.pallas import tpu_sc as plsc

VOCAB = 1_000_000
EMBED_DIM = 64
BATCH = 4096
HIST = 200

NUM_WORKERS = 32
ROWS_PER_W = BATCH // NUM_WORKERS
IDX_PER_W = ROWS_PER_W * HIST
CHUNK = 128
NCHUNKS = IDX_PER_W // CHUNK
FIRE = 8

VBLK = 8000


def _tc_scores_body(w_ref, b_ref, emb_ref, out_ref):
    s = jnp.dot(emb_ref[...], w_ref[...], preferred_element_type=jnp.float32)
    out_ref[...] = (s + b_ref[0, 0]) * (1.0 / HIST)


def _tc_scores(emb_table, w_col, b11):
    return pl.pallas_call(
        _tc_scores_body,
        grid=(VOCAB // VBLK,),
        in_specs=[
            pl.BlockSpec((EMBED_DIM, 1), lambda i: (0, 0)),
            pl.BlockSpec((1, 1), lambda i: (0, 0)),
            pl.BlockSpec((VBLK, EMBED_DIM), lambda i: (i, 0)),
        ],
        out_specs=pl.BlockSpec((VBLK, 1), lambda i: (i, 0)),
        out_shape=jax.ShapeDtypeStruct((VOCAB, 1), jnp.float32),
    )(w_col, b11, emb_table)


def _sc_pool_body(xt_hbm, scores_hbm, out_hbm, xv, gv, ov, sem):
    cid = lax.axis_index("c")
    sid = lax.axis_index("s")
    wid = sid * 2 + cid

    pltpu.sync_copy(xt_hbm.at[:, pl.ds(wid * ROWS_PER_W, ROWS_PER_W)], xv)

    @pl.loop(0, HIST // FIRE)
    def _(i):
        base = i * FIRE
        copies = []
        for u in range(FIRE):
            j = base + u
            copies.append(
                pltpu.async_copy(scores_hbm.at[xv.at[j]], gv.at[j], sem)
            )
        for cp in copies:
            cp.wait()

    for cg in range(ROWS_PER_W // 16):
        @pl.loop(0, HIST, init_carry=jnp.zeros((16,), jnp.float32), unroll=8)
        def acc(j, carry):
            return carry + gv[j, pl.ds(cg * 16, 16)]

        ov[pl.ds(cg * 16, 16)] = acc

    pltpu.sync_copy(ov, out_hbm.at[pl.ds(wid * ROWS_PER_W, ROWS_PER_W)])


@functools.partial(
    pl.kernel,
    out_type=jax.ShapeDtypeStruct((BATCH,), jnp.float32),
    mesh=plsc.VectorSubcoreMesh(core_axis_name="c", subcore_axis_name="s",
                                num_cores=2, num_subcores=16),
    scratch_types=[
        pltpu.VMEM((HIST, ROWS_PER_W), jnp.int32),
        pltpu.VMEM((HIST, ROWS_PER_W), jnp.float32),
        pltpu.VMEM((ROWS_PER_W,), jnp.float32),
        pltpu.SemaphoreType.DMA,
    ],
)
def _sc_pool(xt_hbm, scores_hbm, out_hbm, xv, gv, ov, sem):
    _sc_pool_body(xt_hbm, scores_hbm, out_hbm, xv, gv, ov, sem)


def kernel(x, emb_table, W, b):
    w_col = W.reshape(EMBED_DIM, 1)
    b11 = b.reshape(1, 1)
    scores = _tc_scores(emb_table, w_col, b11).reshape(VOCAB)
    xt = x.astype(jnp.int32).T
    out = _sc_pool(xt, scores)
    return out.reshape(BATCH, 1)


# speedup vs baseline: 1.3886x; 1.3886x over previous
import functools

import jax
import jax.numpy as jnp
from jax import lax
from jax.experimental import pallas as pl
from jax.experimental.pallas import tpu as pltpu
from jax.experimental.pallas import tpu_sc as plsc

VOCAB = 1_000_000
EMBED_DIM = 64
BATCH = 4096
HIST = 200

NUM_WORKERS = 32
ROWS_PER_W = BATCH // NUM_WORKERS
IDX_PER_W = ROWS_PER_W * HIST
CHUNK = 128
NCHUNKS = IDX_PER_W // CHUNK
FIRE = 8

VBLK = 8000


def _tc_scores_body(w_ref, b_ref, emb_ref, out_ref):
    s = lax.dot_general(
        w_ref[...], emb_ref[...],
        dimension_numbers=(((1,), (1,)), ((), ())),
        preferred_element_type=jnp.float32,
    )
    out_ref[...] = ((s + b_ref[0, 0]) * (1.0 / HIST)).reshape(1, 1, VBLK)


def _tc_scores(emb_table, w_row, b11):
    return pl.pallas_call(
        _tc_scores_body,
        grid=(VOCAB // VBLK,),
        in_specs=[
            pl.BlockSpec((1, EMBED_DIM), lambda i: (0, 0)),
            pl.BlockSpec((1, 1), lambda i: (0, 0)),
            pl.BlockSpec((VBLK, EMBED_DIM), lambda i: (i, 0)),
        ],
        out_specs=pl.BlockSpec((1, 1, VBLK), lambda i: (i, 0, 0)),
        out_shape=jax.ShapeDtypeStruct((VOCAB // VBLK, 1, VBLK), jnp.float32),
    )(w_row, b11, emb_table)


def _sc_pool_body(xt_hbm, scores_hbm, out_hbm, xv, gv, ov, sem):
    cid = lax.axis_index("c")
    sid = lax.axis_index("s")
    wid = sid * 2 + cid

    pltpu.sync_copy(xt_hbm.at[:, pl.ds(wid * ROWS_PER_W, ROWS_PER_W)], xv)

    @pl.loop(0, HIST // FIRE)
    def _(i):
        base = i * FIRE
        copies = []
        for u in range(FIRE):
            j = base + u
            copies.append(
                pltpu.async_copy(scores_hbm.at[xv.at[j]], gv.at[j], sem)
            )
        for cp in copies:
            cp.wait()

    for cg in range(ROWS_PER_W // 16):
        @pl.loop(0, HIST, init_carry=jnp.zeros((16,), jnp.float32), unroll=8)
        def acc(j, carry):
            return carry + gv[j, pl.ds(cg * 16, 16)]

        ov[pl.ds(cg * 16, 16)] = acc

    pltpu.sync_copy(ov, out_hbm.at[pl.ds(wid * ROWS_PER_W, ROWS_PER_W)])


@functools.partial(
    pl.kernel,
    out_type=jax.ShapeDtypeStruct((BATCH,), jnp.float32),
    mesh=plsc.VectorSubcoreMesh(core_axis_name="c", subcore_axis_name="s",
                                num_cores=2, num_subcores=16),
    scratch_types=[
        pltpu.VMEM((HIST, ROWS_PER_W), jnp.int32),
        pltpu.VMEM((HIST, ROWS_PER_W), jnp.float32),
        pltpu.VMEM((ROWS_PER_W,), jnp.float32),
        pltpu.SemaphoreType.DMA,
    ],
)
def _sc_pool(xt_hbm, scores_hbm, out_hbm, xv, gv, ov, sem):
    _sc_pool_body(xt_hbm, scores_hbm, out_hbm, xv, gv, ov, sem)


def kernel(x, emb_table, W, b):
    b11 = b.reshape(1, 1)
    scores = _tc_scores(emb_table, W, b11).reshape(VOCAB)
    xt = x.astype(jnp.int32).T
    out = _sc_pool(xt, scores)
    return out.reshape(BATCH, 1)
